# baseline (speedup 1.0000x reference)
import functools

import jax
import jax.numpy as jnp
from jax import lax
from jax.experimental import pallas as pl
from jax.experimental.pallas import tpu as pltpu
from jax.experimental.pallas import tpu_sc as plsc

EMBED = 2048
NC = 2
NS = 16
NW = NC * NS
CHUNK = 8
NBUF = 4


def _emb_body(n_chunks, idx_hbm, table_hbm, out_hbm, idx_v, *scratch):
  bufs = scratch[:NBUF]
  sem_g = scratch[NBUF:2 * NBUF]
  sem_o = scratch[2 * NBUF:3 * NBUF]

  cid = lax.axis_index("c")
  sid = lax.axis_index("s")
  wid = sid * NC + cid
  base = wid * (n_chunks * CHUNK)

  pltpu.sync_copy(idx_hbm.at[wid], idx_v)

  for b in range(NBUF):
    pltpu.async_copy(table_hbm.at[idx_v.at[b]], bufs[b], sem_g[b])

  @pl.loop(0, n_chunks - NBUF, step=NBUF)
  def _(g):
    for b in range(NBUF):
      i = g + b
      pltpu.make_async_copy(table_hbm.at[idx_v.at[i]], bufs[b], sem_g[b]).wait()
      pltpu.async_copy(bufs[b], out_hbm.at[pl.ds(base + i * CHUNK, CHUNK)],
                       sem_o[b])
    for b in range(NBUF):
      i = g + b
      pltpu.make_async_copy(
          bufs[b], out_hbm.at[pl.ds(base + i * CHUNK, CHUNK)], sem_o[b]).wait()
      pltpu.async_copy(table_hbm.at[idx_v.at[i + NBUF]], bufs[b], sem_g[b])

  t0 = n_chunks - NBUF
  for b in range(NBUF):
    i = t0 + b
    pltpu.make_async_copy(table_hbm.at[idx_v.at[i]], bufs[b], sem_g[b]).wait()
    pltpu.async_copy(bufs[b], out_hbm.at[pl.ds(base + i * CHUNK, CHUNK)],
                     sem_o[b])
  for b in range(NBUF):
    i = t0 + b
    pltpu.make_async_copy(
        bufs[b], out_hbm.at[pl.ds(base + i * CHUNK, CHUNK)], sem_o[b]).wait()


def kernel(input_ids, embed_table):
  batch, seq = input_ids.shape
  total = batch * seq
  n_chunks = total // (NW * CHUNK)
  idx = input_ids.reshape(NW, n_chunks, CHUNK).astype(jnp.int32)

  mesh = plsc.VectorSubcoreMesh(core_axis_name="c", subcore_axis_name="s")
  k = pl.kernel(
      functools.partial(_emb_body, n_chunks),
      out_type=jax.ShapeDtypeStruct((total, EMBED), jnp.float32),
      mesh=mesh,
      scratch_types=(
          [pltpu.VMEM((n_chunks, CHUNK), jnp.int32)]
          + [pltpu.VMEM((CHUNK, EMBED), jnp.float32) for _ in range(NBUF)]
          + [pltpu.SemaphoreType.DMA for _ in range(2 * NBUF)]
      ),
  )
  out = k(idx, embed_table)
  return out.reshape(batch, seq, EMBED)
